# baseline (speedup 1.0000x reference)
import functools

import jax
import jax.numpy as jnp
from jax import lax
from jax.experimental import pallas as pl
from jax.experimental.pallas import tpu as pltpu
from jax.experimental.pallas import tpu_sc as plsc

N = 10000
E = 320000
D_IN = 128
F = 8

NC, NS = 2, 16
NW = NC * NS
N_PAD = 10240
E_PAD = 327680
CHUNK = 128
CH = E_PAD // (NW * CHUNK)
NDEEP = 16
ROWS = N_PAD // NS

_mesh = plsc.VectorSubcoreMesh(
    core_axis_name="c", subcore_axis_name="s", num_cores=NC, num_subcores=NS
)
_sc_params = pltpu.CompilerParams(use_tc_tiling_on_sc=False)


@functools.partial(
    pl.kernel,
    out_type=jax.ShapeDtypeStruct((NC, N_PAD), jnp.float32),
    mesh=_mesh,
    scratch_types=[
        pltpu.VMEM((CH, CHUNK), jnp.int32),
        pltpu.VMEM((CHUNK,), jnp.float32),
        pltpu.VMEM_SHARED((N_PAD,), jnp.float32),
        pltpu.SemaphoreType.DMA,
    ],
    compiler_params=_sc_params,
)
def _degree_pass(dst_hbm, ones_hbm, zero_hbm, out_hbm, dst_v, msg_v, acc, sem):
    cid = lax.axis_index("c")
    sid = lax.axis_index("s")
    wid = cid * NS + sid
    pltpu.sync_copy(dst_hbm.at[wid], dst_v)
    pltpu.sync_copy(ones_hbm, msg_v)
    r0 = sid * ROWS
    pltpu.sync_copy(zero_hbm.at[pl.ds(r0, ROWS)], acc.at[pl.ds(r0, ROWS)])
    plsc.subcore_barrier()

    def fire(j, carry):
        pltpu.async_copy(msg_v, acc.at[dst_v.at[j]], sem, add=True)
        return carry

    def drain(j, carry):
        pltpu.make_async_copy(msg_v, acc.at[dst_v.at[j]], sem).wait()
        return carry

    lax.fori_loop(0, CH, fire, 0)
    lax.fori_loop(0, CH, drain, 0)
    plsc.subcore_barrier()
    pltpu.sync_copy(acc.at[pl.ds(r0, ROWS)], out_hbm.at[cid, pl.ds(r0, ROWS)])


@functools.partial(
    pl.kernel,
    out_type=jax.ShapeDtypeStruct((NC, N_PAD, F), jnp.float32),
    mesh=_mesh,
    scratch_types=[
        pltpu.VMEM((CH, CHUNK), jnp.int32),
        pltpu.VMEM((CH, CHUNK), jnp.int32),
        pltpu.VMEM((CH, CHUNK, F), jnp.float32),
        pltpu.VMEM_SHARED((N_PAD, F), jnp.float32),
        pltpu.SemaphoreType.DMA,
    ] + [pltpu.SemaphoreType.DMA] * NDEEP,
    compiler_params=_sc_params,
)
def _message_pass(src_hbm, dst_hbm, y_hbm, zero_hbm, out_hbm,
                  src_v, dst_v, msg_v, acc, ssem, *gsems):
    cid = lax.axis_index("c")
    sid = lax.axis_index("s")
    wid = cid * NS + sid
    pltpu.sync_copy(src_hbm.at[wid], src_v)
    pltpu.sync_copy(dst_hbm.at[wid], dst_v)
    r0 = sid * ROWS
    pltpu.sync_copy(zero_hbm.at[pl.ds(r0, ROWS)], acc.at[pl.ds(r0, ROWS)])
    plsc.subcore_barrier()

    y_core = y_hbm.at[cid]
    for b in range(NDEEP):
        pltpu.async_copy(y_core.at[src_v.at[b]], msg_v.at[b], gsems[b])

    def group(g, carry):
        for b in range(NDEEP):
            j = g * NDEEP + b
            pltpu.make_async_copy(y_core.at[src_v.at[j]], msg_v.at[j],
                                  gsems[b]).wait()
            pltpu.async_copy(msg_v.at[j], acc.at[dst_v.at[j]], ssem, add=True)

            @pl.when(g < CH // NDEEP - 1)
            def _():
                pltpu.async_copy(y_core.at[src_v.at[j + NDEEP]],
                                 msg_v.at[j + NDEEP], gsems[b])
        return carry

    lax.fori_loop(0, CH // NDEEP, group, 0)

    def drain_scatter(j, carry):
        pltpu.make_async_copy(msg_v.at[j], acc.at[dst_v.at[j]], ssem).wait()
        return carry

    lax.fori_loop(0, CH, drain_scatter, 0)
    plsc.subcore_barrier()
    pltpu.sync_copy(acc.at[pl.ds(r0, ROWS)], out_hbm.at[cid, pl.ds(r0, ROWS)])


_RB = 2048


def _dinv_of(deg_ref):
    deg = deg_ref[0] + deg_ref[1] + 1.0
    return lax.rsqrt(deg)


def _xw_body(x_ref, w_ref, o_ref):
    o_ref[...] = jnp.dot(x_ref[...], w_ref[...],
                         preferred_element_type=jnp.float32)


def _y1_body(deg_ref, xw_ref, y_ref):
    dinv = _dinv_of(deg_ref)
    y = dinv[:, None] * xw_ref[...]
    y_ref[0] = y
    y_ref[1] = y


def _layer1_body(deg_ref, agg_ref, y1_ref, b1_ref, w2_ref, y2_ref):
    dinv = _dinv_of(deg_ref)
    s = agg_ref[0] + agg_ref[1] + y1_ref[0]
    h = jnp.maximum(dinv[:, None] * s + b1_ref[...], 0.0)
    y2 = dinv[:, None] * jnp.dot(
        h, w2_ref[...], preferred_element_type=jnp.float32)
    y2_ref[0] = y2
    y2_ref[1] = y2


def _layer2_body(deg_ref, agg_ref, y2_ref, b2_ref, out_ref):
    dinv = _dinv_of(deg_ref)
    s = agg_ref[0] + agg_ref[1] + y2_ref[0]
    out_ref[...] = dinv[:, None] * s + b2_ref[...]


def _tc_call(body, in_specs, out_shape, out_spec):
    return pl.pallas_call(
        body,
        grid=(N_PAD // _RB,),
        in_specs=in_specs,
        out_specs=out_spec,
        out_shape=jax.ShapeDtypeStruct(out_shape, jnp.float32),
    )


_deg_spec = pl.BlockSpec((NC, _RB), lambda i: (0, i))
_dup_spec = pl.BlockSpec((NC, _RB, F), lambda i: (0, i, 0))
_row_spec = pl.BlockSpec((_RB, F), lambda i: (i, 0))
_vec_spec = pl.BlockSpec((1, F), lambda i: (0, 0))


def kernel(x, edge_index, W1, b1, W2, b2):
    src = edge_index[0]
    dst = edge_index[1]
    pad_e = jnp.full((E_PAD - E,), N, dtype=jnp.int32)
    src3 = jnp.concatenate([src, pad_e]).reshape(NW, CH, CHUNK)
    dst3 = jnp.concatenate([dst, pad_e]).reshape(NW, CH, CHUNK)

    x_pad = jnp.zeros((N_PAD, D_IN), x.dtype).at[:N].set(x)
    zero_rows = jnp.zeros((N_PAD, F), jnp.float32)
    zero_deg = jnp.zeros((N_PAD,), jnp.float32)
    ones_deg = jnp.ones((CHUNK,), jnp.float32)
    w2p = jnp.zeros((F, F), W2.dtype).at[:, : W2.shape[1]].set(W2)
    b1r = b1.reshape(1, F)
    b2r = jnp.zeros((1, F), b2.dtype).at[0, : b2.shape[0]].set(b2)

    xw = _tc_call(
        _xw_body,
        [pl.BlockSpec((_RB, D_IN), lambda i: (i, 0)),
         pl.BlockSpec((D_IN, F), lambda i: (0, 0))],
        (N_PAD, F), _row_spec,
    )(x_pad, W1)

    degp = _degree_pass(dst3, ones_deg, zero_deg)

    y1 = _tc_call(
        _y1_body, [_deg_spec, _row_spec], (NC, N_PAD, F), _dup_spec,
    )(degp, xw)

    agg1 = _message_pass(src3, dst3, y1, zero_rows)

    y2 = _tc_call(
        _layer1_body,
        [_deg_spec, _dup_spec, _dup_spec, _vec_spec,
         pl.BlockSpec((F, F), lambda i: (0, 0))],
        (NC, N_PAD, F), _dup_spec,
    )(degp, agg1, y1, b1r, w2p)

    agg2 = _message_pass(src3, dst3, y2, zero_rows)

    out = _tc_call(
        _layer2_body,
        [_deg_spec, _dup_spec, _dup_spec, _vec_spec],
        (N_PAD, F), _row_spec,
    )(degp, agg2, y2, b2r)

    return out[:N, : W2.shape[1]]


# speedup vs baseline: 1.4677x; 1.4677x over previous
import functools

import jax
import jax.numpy as jnp
from jax import lax
from jax.experimental import pallas as pl
from jax.experimental.pallas import tpu as pltpu
from jax.experimental.pallas import tpu_sc as plsc

N = 10000
E = 320000
D_IN = 128
F = 8

NC, NS = 2, 16
NW = NC * NS
N_PAD = 10240
E_PAD = 327680
CHUNK = 128
CH = E_PAD // (NW * CHUNK)
NDEEP = 16
ROWS = N_PAD // NS

_mesh = plsc.VectorSubcoreMesh(
    core_axis_name="c", subcore_axis_name="s", num_cores=NC, num_subcores=NS
)
_sc_params = pltpu.CompilerParams(use_tc_tiling_on_sc=False)


@functools.partial(
    pl.kernel,
    out_type=jax.ShapeDtypeStruct((NC, N_PAD), jnp.float32),
    mesh=_mesh,
    scratch_types=[
        pltpu.VMEM((CH, CHUNK), jnp.int32),
        pltpu.VMEM((CHUNK,), jnp.float32),
        pltpu.VMEM_SHARED((N_PAD,), jnp.float32),
        pltpu.SemaphoreType.DMA,
    ],
    compiler_params=_sc_params,
)
def _degree_pass(dst_hbm, ones_hbm, zero_hbm, out_hbm, dst_v, msg_v, acc, sem):
    cid = lax.axis_index("c")
    sid = lax.axis_index("s")
    wid = cid * NS + sid
    pltpu.sync_copy(dst_hbm.at[wid], dst_v)
    pltpu.sync_copy(ones_hbm, msg_v)
    r0 = sid * ROWS
    pltpu.sync_copy(zero_hbm.at[pl.ds(r0, ROWS)], acc.at[pl.ds(r0, ROWS)])
    plsc.subcore_barrier()

    def fire(j, carry):
        pltpu.async_copy(msg_v, acc.at[dst_v.at[j]], sem, add=True)
        return carry

    def drain(j, carry):
        pltpu.make_async_copy(msg_v, acc.at[dst_v.at[j]], sem).wait()
        return carry

    lax.fori_loop(0, CH, fire, 0)
    lax.fori_loop(0, CH, drain, 0)
    plsc.subcore_barrier()
    pltpu.sync_copy(acc.at[pl.ds(r0, ROWS)], out_hbm.at[cid, pl.ds(r0, ROWS)])


@functools.partial(
    pl.kernel,
    out_type=jax.ShapeDtypeStruct((NC, N_PAD, F), jnp.float32),
    mesh=_mesh,
    scratch_types=[
        pltpu.VMEM((CH, CHUNK), jnp.int32),
        pltpu.VMEM((CH, CHUNK), jnp.int32),
        pltpu.VMEM((CH, CHUNK, F), jnp.float32),
        pltpu.VMEM_SHARED((N_PAD, F), jnp.float32),
        pltpu.VMEM_SHARED((N_PAD, F), jnp.float32),
        pltpu.SemaphoreType.DMA,
    ] + [pltpu.SemaphoreType.DMA] * NDEEP,
    compiler_params=_sc_params,
)
def _message_pass(src_hbm, dst_hbm, y_hbm, zero_hbm, out_hbm,
                  src_v, dst_v, msg_v, acc, y_spm, ssem, *gsems):
    cid = lax.axis_index("c")
    sid = lax.axis_index("s")
    wid = cid * NS + sid
    pltpu.sync_copy(src_hbm.at[wid], src_v)
    pltpu.sync_copy(dst_hbm.at[wid], dst_v)
    r0 = sid * ROWS
    pltpu.sync_copy(y_hbm.at[pl.ds(r0, ROWS)], y_spm.at[pl.ds(r0, ROWS)])
    pltpu.sync_copy(zero_hbm.at[pl.ds(r0, ROWS)], acc.at[pl.ds(r0, ROWS)])
    plsc.subcore_barrier()

    y_core = y_spm
    for b in range(NDEEP):
        pltpu.async_copy(y_core.at[src_v.at[b]], msg_v.at[b], gsems[b])

    def group(g, carry):
        for b in range(NDEEP):
            j = g * NDEEP + b
            pltpu.make_async_copy(y_core.at[src_v.at[j]], msg_v.at[j],
                                  gsems[b]).wait()
            pltpu.async_copy(msg_v.at[j], acc.at[dst_v.at[j]], ssem, add=True)

            @pl.when(g < CH // NDEEP - 1)
            def _():
                pltpu.async_copy(y_core.at[src_v.at[j + NDEEP]],
                                 msg_v.at[j + NDEEP], gsems[b])
        return carry

    lax.fori_loop(0, CH // NDEEP, group, 0)

    def drain_scatter(j, carry):
        pltpu.make_async_copy(msg_v.at[j], acc.at[dst_v.at[j]], ssem).wait()
        return carry

    lax.fori_loop(0, CH, drain_scatter, 0)
    plsc.subcore_barrier()
    pltpu.sync_copy(acc.at[pl.ds(r0, ROWS)], out_hbm.at[cid, pl.ds(r0, ROWS)])


_RB = 2048


def _dinv_of(deg_ref):
    deg = deg_ref[0] + deg_ref[1] + 1.0
    return lax.rsqrt(deg)


def _y1_body(deg_ref, x_ref, w_ref, y_ref):
    dinv = _dinv_of(deg_ref)
    xw = jnp.dot(x_ref[...], w_ref[...], preferred_element_type=jnp.float32)
    y_ref[...] = dinv[:, None] * xw


def _layer1_body(deg_ref, agg_ref, y1_ref, b1_ref, w2_ref, y2_ref):
    dinv = _dinv_of(deg_ref)
    s = agg_ref[0] + agg_ref[1] + y1_ref[...]
    h = jnp.maximum(dinv[:, None] * s + b1_ref[...], 0.0)
    y2_ref[...] = dinv[:, None] * jnp.dot(
        h, w2_ref[...], preferred_element_type=jnp.float32)


def _layer2_body(deg_ref, agg_ref, y2_ref, b2_ref, out_ref):
    dinv = _dinv_of(deg_ref)
    s = agg_ref[0] + agg_ref[1] + y2_ref[...]
    out_ref[...] = dinv[:, None] * s + b2_ref[...]


def _tc_call(body, in_specs, out_shape, out_spec):
    return pl.pallas_call(
        body,
        grid=(N_PAD // _RB,),
        in_specs=in_specs,
        out_specs=out_spec,
        out_shape=jax.ShapeDtypeStruct(out_shape, jnp.float32),
    )


_deg_spec = pl.BlockSpec((NC, _RB), lambda i: (0, i))
_dup_spec = pl.BlockSpec((NC, _RB, F), lambda i: (0, i, 0))
_row_spec = pl.BlockSpec((_RB, F), lambda i: (i, 0))
_vec_spec = pl.BlockSpec((1, F), lambda i: (0, 0))


def kernel(x, edge_index, W1, b1, W2, b2):
    src = edge_index[0]
    dst = edge_index[1]
    pad_e = jnp.full((E_PAD - E,), N, dtype=jnp.int32)
    src3 = jnp.concatenate([src, pad_e]).reshape(NW, CH, CHUNK)
    dst3 = jnp.concatenate([dst, pad_e]).reshape(NW, CH, CHUNK)

    x_pad = jnp.zeros((N_PAD, D_IN), x.dtype).at[:N].set(x)
    zero_rows = jnp.zeros((N_PAD, F), jnp.float32)
    zero_deg = jnp.zeros((N_PAD,), jnp.float32)
    ones_deg = jnp.ones((CHUNK,), jnp.float32)
    w2p = jnp.zeros((F, F), W2.dtype).at[:, : W2.shape[1]].set(W2)
    b1r = b1.reshape(1, F)
    b2r = jnp.zeros((1, F), b2.dtype).at[0, : b2.shape[0]].set(b2)

    degp = _degree_pass(dst3, ones_deg, zero_deg)

    y1 = _tc_call(
        _y1_body,
        [_deg_spec,
         pl.BlockSpec((_RB, D_IN), lambda i: (i, 0)),
         pl.BlockSpec((D_IN, F), lambda i: (0, 0))],
        (N_PAD, F), _row_spec,
    )(degp, x_pad, W1)

    agg1 = _message_pass(src3, dst3, y1, zero_rows)

    y2 = _tc_call(
        _layer1_body,
        [_deg_spec, _dup_spec, _row_spec, _vec_spec,
         pl.BlockSpec((F, F), lambda i: (0, 0))],
        (N_PAD, F), _row_spec,
    )(degp, agg1, y1, b1r, w2p)

    agg2 = _message_pass(src3, dst3, y2, zero_rows)

    out = _tc_call(
        _layer2_body,
        [_deg_spec, _dup_spec, _row_spec, _vec_spec],
        (N_PAD, F), _row_spec,
    )(degp, agg2, y2, b2r)

    return out[:N, : W2.shape[1]]


# speedup vs baseline: 1.5383x; 1.0481x over previous
import functools

import jax
import jax.numpy as jnp
from jax import lax
from jax.experimental import pallas as pl
from jax.experimental.pallas import tpu as pltpu
from jax.experimental.pallas import tpu_sc as plsc

N = 10000
E = 320000
D_IN = 128
F = 8

NC, NS = 2, 16
NW = NC * NS
N_PAD = 10240
E_PAD = 327680
CHUNK = 128
CH = E_PAD // (NW * CHUNK)
NDEEP = 16
ROWS = N_PAD // NS

_mesh = plsc.VectorSubcoreMesh(
    core_axis_name="c", subcore_axis_name="s", num_cores=NC, num_subcores=NS
)
_sc_params = pltpu.CompilerParams(use_tc_tiling_on_sc=False)


@functools.partial(
    pl.kernel,
    out_type=jax.ShapeDtypeStruct((NC, N_PAD), jnp.float32),
    mesh=_mesh,
    scratch_types=[
        pltpu.VMEM((CH, CHUNK), jnp.int32),
        pltpu.VMEM((CHUNK,), jnp.float32),
        pltpu.VMEM_SHARED((N_PAD,), jnp.float32),
        pltpu.SemaphoreType.DMA,
    ],
    compiler_params=_sc_params,
)
def _degree_pass(dst_hbm, ones_hbm, zero_hbm, out_hbm, dst_v, msg_v, acc, sem):
    cid = lax.axis_index("c")
    sid = lax.axis_index("s")
    wid = cid * NS + sid
    r0 = sid * ROWS
    pltpu.async_copy(dst_hbm.at[wid], dst_v, sem)
    pltpu.async_copy(ones_hbm, msg_v, sem)
    pltpu.async_copy(zero_hbm.at[pl.ds(r0, ROWS)], acc.at[pl.ds(r0, ROWS)], sem)
    pltpu.make_async_copy(dst_hbm.at[wid], dst_v, sem).wait()
    pltpu.make_async_copy(ones_hbm, msg_v, sem).wait()
    pltpu.make_async_copy(
        zero_hbm.at[pl.ds(r0, ROWS)], acc.at[pl.ds(r0, ROWS)], sem).wait()
    plsc.subcore_barrier()

    def fire(j, carry):
        pltpu.async_copy(msg_v, acc.at[dst_v.at[j]], sem, add=True)
        return carry

    def drain(j, carry):
        pltpu.make_async_copy(msg_v, acc.at[dst_v.at[j]], sem).wait()
        return carry

    lax.fori_loop(0, CH, fire, 0)
    lax.fori_loop(0, CH, drain, 0)
    plsc.subcore_barrier()
    pltpu.sync_copy(acc.at[pl.ds(r0, ROWS)], out_hbm.at[cid, pl.ds(r0, ROWS)])


@functools.partial(
    pl.kernel,
    out_type=jax.ShapeDtypeStruct((NC, N_PAD, F), jnp.float32),
    mesh=_mesh,
    scratch_types=[
        pltpu.VMEM((CH, CHUNK), jnp.int32),
        pltpu.VMEM((CH, CHUNK), jnp.int32),
        pltpu.VMEM((CH, CHUNK, F), jnp.float32),
        pltpu.VMEM_SHARED((N_PAD, F), jnp.float32),
        pltpu.VMEM_SHARED((N_PAD, F), jnp.float32),
        pltpu.SemaphoreType.DMA,
    ] + [pltpu.SemaphoreType.DMA] * NDEEP,
    compiler_params=_sc_params,
)
def _message_pass(src_hbm, dst_hbm, y_hbm, zero_hbm, out_hbm,
                  src_v, dst_v, msg_v, acc, y_spm, ssem, *gsems):
    cid = lax.axis_index("c")
    sid = lax.axis_index("s")
    wid = cid * NS + sid
    r0 = sid * ROWS
    pltpu.async_copy(src_hbm.at[wid], src_v, ssem)
    pltpu.async_copy(dst_hbm.at[wid], dst_v, ssem)
    pltpu.async_copy(y_hbm.at[pl.ds(r0, ROWS)], y_spm.at[pl.ds(r0, ROWS)], ssem)
    pltpu.async_copy(zero_hbm.at[pl.ds(r0, ROWS)], acc.at[pl.ds(r0, ROWS)], ssem)
    pltpu.make_async_copy(src_hbm.at[wid], src_v, ssem).wait()
    pltpu.make_async_copy(dst_hbm.at[wid], dst_v, ssem).wait()
    pltpu.make_async_copy(
        y_hbm.at[pl.ds(r0, ROWS)], y_spm.at[pl.ds(r0, ROWS)], ssem).wait()
    pltpu.make_async_copy(
        zero_hbm.at[pl.ds(r0, ROWS)], acc.at[pl.ds(r0, ROWS)], ssem).wait()
    plsc.subcore_barrier()

    y_core = y_spm
    for b in range(NDEEP):
        pltpu.async_copy(y_core.at[src_v.at[b]], msg_v.at[b], gsems[b])

    def group(g, carry):
        for b in range(NDEEP):
            j = g * NDEEP + b
            pltpu.make_async_copy(y_core.at[src_v.at[j]], msg_v.at[j],
                                  gsems[b]).wait()
            pltpu.async_copy(msg_v.at[j], acc.at[dst_v.at[j]], ssem, add=True)

            @pl.when(g < CH // NDEEP - 1)
            def _():
                pltpu.async_copy(y_core.at[src_v.at[j + NDEEP]],
                                 msg_v.at[j + NDEEP], gsems[b])
        return carry

    lax.fori_loop(0, CH // NDEEP, group, 0)

    def drain_scatter(j, carry):
        pltpu.make_async_copy(msg_v.at[j], acc.at[dst_v.at[j]], ssem).wait()
        return carry

    lax.fori_loop(0, CH, drain_scatter, 0)
    plsc.subcore_barrier()
    pltpu.sync_copy(acc.at[pl.ds(r0, ROWS)], out_hbm.at[cid, pl.ds(r0, ROWS)])


_RB = 2048


def _dinv_of(deg_ref):
    deg = deg_ref[0] + deg_ref[1] + 1.0
    return lax.rsqrt(deg)


def _y1_body(deg_ref, x_ref, w_ref, y_ref):
    dinv = _dinv_of(deg_ref)
    xw = jnp.dot(x_ref[...], w_ref[...], preferred_element_type=jnp.float32)
    y_ref[...] = dinv[:, None] * xw


def _layer1_body(deg_ref, agg_ref, y1_ref, b1_ref, w2_ref, y2_ref):
    dinv = _dinv_of(deg_ref)
    s = agg_ref[0] + agg_ref[1] + y1_ref[...]
    h = jnp.maximum(dinv[:, None] * s + b1_ref[...], 0.0)
    y2_ref[...] = dinv[:, None] * jnp.dot(
        h, w2_ref[...], preferred_element_type=jnp.float32)


def _layer2_body(deg_ref, agg_ref, y2_ref, b2_ref, out_ref):
    dinv = _dinv_of(deg_ref)
    s = agg_ref[0] + agg_ref[1] + y2_ref[...]
    out_ref[...] = dinv[:, None] * s + b2_ref[...]


def _tc_call(body, in_specs, out_shape, out_spec):
    return pl.pallas_call(
        body,
        grid=(N_PAD // _RB,),
        in_specs=in_specs,
        out_specs=out_spec,
        out_shape=jax.ShapeDtypeStruct(out_shape, jnp.float32),
    )


_deg_spec = pl.BlockSpec((NC, _RB), lambda i: (0, i))
_dup_spec = pl.BlockSpec((NC, _RB, F), lambda i: (0, i, 0))
_row_spec = pl.BlockSpec((_RB, F), lambda i: (i, 0))
_vec_spec = pl.BlockSpec((1, F), lambda i: (0, 0))


def kernel(x, edge_index, W1, b1, W2, b2):
    src = edge_index[0]
    dst = edge_index[1]
    pad_e = jnp.full((E_PAD - E,), N, dtype=jnp.int32)
    src3 = jnp.concatenate([src, pad_e]).reshape(NW, CH, CHUNK)
    dst3 = jnp.concatenate([dst, pad_e]).reshape(NW, CH, CHUNK)

    x_pad = jnp.zeros((N_PAD, D_IN), x.dtype).at[:N].set(x)
    zero_rows = jnp.zeros((N_PAD, F), jnp.float32)
    zero_deg = jnp.zeros((N_PAD,), jnp.float32)
    ones_deg = jnp.ones((CHUNK,), jnp.float32)
    w2p = jnp.zeros((F, F), W2.dtype).at[:, : W2.shape[1]].set(W2)
    b1r = b1.reshape(1, F)
    b2r = jnp.zeros((1, F), b2.dtype).at[0, : b2.shape[0]].set(b2)

    degp = _degree_pass(dst3, ones_deg, zero_deg)

    y1 = _tc_call(
        _y1_body,
        [_deg_spec,
         pl.BlockSpec((_RB, D_IN), lambda i: (i, 0)),
         pl.BlockSpec((D_IN, F), lambda i: (0, 0))],
        (N_PAD, F), _row_spec,
    )(degp, x_pad, W1)

    agg1 = _message_pass(src3, dst3, y1, zero_rows)

    y2 = _tc_call(
        _layer1_body,
        [_deg_spec, _dup_spec, _row_spec, _vec_spec,
         pl.BlockSpec((F, F), lambda i: (0, 0))],
        (N_PAD, F), _row_spec,
    )(degp, agg1, y1, b1r, w2p)

    agg2 = _message_pass(src3, dst3, y2, zero_rows)

    out = _tc_call(
        _layer2_body,
        [_deg_spec, _dup_spec, _row_spec, _vec_spec],
        (N_PAD, F), _row_spec,
    )(degp, agg2, y2, b2r)

    return out[:N, : W2.shape[1]]
